# baseline (speedup 1.0000x reference)
import functools

import jax
import jax.numpy as jnp
from jax import lax
from jax.experimental import pallas as pl
from jax.experimental.pallas import tpu as pltpu
from jax.experimental.pallas import tpu_sc as plsc

_B = 8192
_C = 8192
_D = 32
_BETA = 0.25
_TB = 256
_NSTEPS = _B // _TB
_CHUNK = 2048

_NC, _NS = 2, 16
_NW = _NC * _NS
_BPW = _B // _NW


def _round_bf16(v):
    u = lax.bitcast_convert_type(v, jnp.uint32)
    r = (u + jnp.uint32(0x7FFF) + ((u >> jnp.uint32(16)) & jnp.uint32(1))) \
        & jnp.uint32(0xFFFF0000)
    return lax.bitcast_convert_type(r, jnp.float32)


def _argmin_body(x_ref, codes_ref, xsq_ref, csq_ref, idx_ref, loss_ref):
    i = pl.program_id(0)
    x = x_ref[...]
    x_sq = xsq_ref[...]
    xb = x.astype(jnp.bfloat16)
    acc_r = acc_v = acc_i = None
    for c in range(_C // _CHUNK):
        lo = c * _CHUNK
        codes_c = codes_ref[lo:lo + _CHUNK, :]
        c_sq_c = csq_ref[:, lo:lo + _CHUNK]
        dot = lax.dot_general(xb, codes_c.astype(jnp.bfloat16),
                              (((1,), (1,)), ((), ())),
                              preferred_element_type=jnp.float32)
        dist = (x_sq - 2.0 * dot) + c_sq_c
        m = jnp.min(dist, axis=-1, keepdims=True)
        cidx = lax.broadcasted_iota(jnp.int32, dist.shape, 1) + lo
        i_c = jnp.min(jnp.where(dist == m, cidx, _C),
                      axis=-1, keepdims=True)
        if c == 0:
            acc_r, acc_v, acc_i = _round_bf16(m), m, i_c
        else:
            win = m < acc_r
            acc_r = jnp.where(win, _round_bf16(m), acc_r)
            acc_v = jnp.where(win, m, acc_v)
            acc_i = jnp.where(win, i_c, acc_i)
    idx_ref[...] = acc_i
    sel = acc_v

    @pl.when(i == 0)
    def _():
        loss_ref[...] = jnp.zeros((1, 1), jnp.float32)

    loss_ref[...] += jnp.sum(sel).reshape(1, 1)

    @pl.when(i == _NSTEPS - 1)
    def _():
        loss_ref[...] = loss_ref[...] * ((1.0 + _BETA) / _B)


_argmin_kernel = pl.pallas_call(
    _argmin_body,
    grid=(_NSTEPS,),
    in_specs=[
        pl.BlockSpec((_TB, _D), lambda i: (i, 0)),
        pl.BlockSpec((_C, _D), lambda i: (0, 0)),
        pl.BlockSpec((_TB, 1), lambda i: (i, 0)),
        pl.BlockSpec((1, _C), lambda i: (0, 0)),
    ],
    out_specs=[
        pl.BlockSpec((_TB, 1), lambda i: (i, 0)),
        pl.BlockSpec((1, 1), lambda i: (0, 0)),
    ],
    out_shape=[
        jax.ShapeDtypeStruct((_B, 1), jnp.int32),
        jax.ShapeDtypeStruct((1, 1), jnp.float32),
    ],
    compiler_params=pltpu.CompilerParams(
        dimension_semantics=("arbitrary",),
    ),
)


_DPAD = 128


@functools.cache
def _make_sc_gather():
    mesh = plsc.VectorSubcoreMesh(core_axis_name="c", subcore_axis_name="s")

    @functools.partial(
        pl.kernel,
        mesh=mesh,
        out_type=jax.ShapeDtypeStruct((_B, _DPAD), jnp.float32),
        scratch_types=[
            pltpu.VMEM((_BPW,), jnp.int32),
            pltpu.VMEM((_BPW, _DPAD), jnp.float32),
            pltpu.SemaphoreType.DMA,
        ],
    )
    def _sc_gather(table_hbm, idx_hbm, out_hbm, idx_v, rows_v, sem):
        wid = lax.axis_index("s") * _NC + lax.axis_index("c")
        base = wid * _BPW
        pltpu.sync_copy(idx_hbm.at[pl.ds(base, _BPW)], idx_v)
        pltpu.async_copy(table_hbm.at[idx_v], rows_v, sem).wait()
        pltpu.sync_copy(rows_v, out_hbm.at[pl.ds(base, _BPW)])

    return _sc_gather


def kernel(x, codes):
    codes2d = codes[0]
    x_sq = jnp.sum(x * x, axis=-1, keepdims=True)
    c_sq = jnp.sum(codes2d * codes2d, axis=-1)[None, :]
    idx2, loss11 = _argmin_kernel(x, codes2d, x_sq, c_sq)
    indices = idx2.reshape(_B)
    codes_pad = jnp.pad(codes2d, ((0, 0), (0, _DPAD - _D)))
    quantized = _make_sc_gather()(codes_pad, indices)[:, :_D]
    loss = loss11.reshape(())
    return quantized, indices, loss


# speedup vs baseline: 1.2357x; 1.2357x over previous
import functools

import jax
import jax.numpy as jnp
from jax import lax
from jax.experimental import pallas as pl
from jax.experimental.pallas import tpu as pltpu
from jax.experimental.pallas import tpu_sc as plsc

_B = 8192
_C = 8192
_D = 32
_BETA = 0.25
_TB = 256
_NSTEPS = _B // _TB
_CHUNK = 2048

_NC, _NS = 2, 16
_NW = _NC * _NS
_BPW = _B // _NW


def _round_bf16(v):
    u = lax.bitcast_convert_type(v, jnp.uint32)
    r = (u + jnp.uint32(0x7FFF) + ((u >> jnp.uint32(16)) & jnp.uint32(1))) \
        & jnp.uint32(0xFFFF0000)
    return lax.bitcast_convert_type(r, jnp.float32)


def _argmin_body(x_ref, codesbf_ref, xsq_ref, csq_ref, lane_ref,
                 idx_ref, loss_ref):
    i = pl.program_id(0)
    x_sq = xsq_ref[...]
    xb = x_ref[...].astype(jnp.bfloat16)
    lane = lane_ref[...]
    acc_r = acc_v = acc_i = None
    for c in range(_C // _CHUNK):
        lo = c * _CHUNK
        codes_c = codesbf_ref[lo:lo + _CHUNK, :]
        dot2 = lax.dot_general(xb, codes_c, (((1,), (1,)), ((), ())),
                               preferred_element_type=jnp.float32)
        rv = rj = None
        for s in range(_CHUNK // 128):
            sl = lo + s * 128
            d = (x_sq + dot2[:, s * 128:(s + 1) * 128]) \
                + csq_ref[:, sl:sl + 128]
            jf = lane + jnp.float32(sl)
            if s == 0:
                rv = d
                rj = jf + jnp.zeros_like(d)
            else:
                lt = d < rv
                rv = jnp.where(lt, d, rv)
                rj = jnp.where(lt, jf + jnp.zeros_like(d), rj)
        m = jnp.min(rv, axis=-1, keepdims=True)
        i_c = jnp.min(jnp.where(rv == m, rj, jnp.float32(_C)),
                      axis=-1, keepdims=True)
        if c == 0:
            acc_r, acc_v, acc_i = _round_bf16(m), m, i_c
        else:
            win = m < acc_r
            acc_r = jnp.where(win, _round_bf16(m), acc_r)
            acc_v = jnp.where(win, m, acc_v)
            acc_i = jnp.where(win, i_c, acc_i)
    idx_ref[...] = acc_i.astype(jnp.int32)
    sel = acc_v

    @pl.when(i == 0)
    def _():
        loss_ref[...] = jnp.zeros((1, 1), jnp.float32)

    loss_ref[...] += jnp.sum(sel).reshape(1, 1)

    @pl.when(i == _NSTEPS - 1)
    def _():
        loss_ref[...] = loss_ref[...] * ((1.0 + _BETA) / _B)


_argmin_kernel = pl.pallas_call(
    _argmin_body,
    grid=(_NSTEPS,),
    in_specs=[
        pl.BlockSpec((_TB, _D), lambda i: (i, 0)),
        pl.BlockSpec((_C, _D), lambda i: (0, 0)),
        pl.BlockSpec((_TB, 1), lambda i: (i, 0)),
        pl.BlockSpec((1, _C), lambda i: (0, 0)),
        pl.BlockSpec((1, 128), lambda i: (0, 0)),
    ],
    out_specs=[
        pl.BlockSpec((_TB, 1), lambda i: (i, 0)),
        pl.BlockSpec((1, 1), lambda i: (0, 0)),
    ],
    out_shape=[
        jax.ShapeDtypeStruct((_B, 1), jnp.int32),
        jax.ShapeDtypeStruct((1, 1), jnp.float32),
    ],
    compiler_params=pltpu.CompilerParams(
        dimension_semantics=("arbitrary",),
    ),
)


_DPAD = 128


@functools.cache
def _make_sc_gather():
    mesh = plsc.VectorSubcoreMesh(core_axis_name="c", subcore_axis_name="s")

    @functools.partial(
        pl.kernel,
        mesh=mesh,
        out_type=jax.ShapeDtypeStruct((_B, _D), jnp.float32),
        scratch_types=[
            pltpu.VMEM((_BPW,), jnp.int32),
            pltpu.VMEM((_BPW, _D), jnp.float32),
            pltpu.SemaphoreType.DMA,
        ],
        compiler_params=pltpu.CompilerParams(use_tc_tiling_on_sc=False),
    )
    def _sc_gather(table_hbm, idx_hbm, out_hbm, idx_v, rows_v, sem):
        wid = lax.axis_index("s") * _NC + lax.axis_index("c")
        base = wid * _BPW
        pltpu.sync_copy(idx_hbm.at[pl.ds(base, _BPW)], idx_v)
        pltpu.async_copy(table_hbm.at[idx_v], rows_v, sem).wait()
        pltpu.sync_copy(rows_v, out_hbm.at[pl.ds(base, _BPW)])

    return _sc_gather


def kernel(x, codes):
    codes2d = codes[0]
    x_sq = jnp.sum(x * x, axis=-1, keepdims=True)
    c_sq = jnp.sum(codes2d * codes2d, axis=-1)[None, :]
    codes_bf = (codes2d * jnp.float32(-2.0)).astype(jnp.bfloat16)
    lane = lax.iota(jnp.float32, 128)[None, :]
    idx2, loss11 = _argmin_kernel(x, codes_bf, x_sq, c_sq, lane)
    indices = idx2.reshape(_B)
    quantized = _make_sc_gather()(codes2d, indices)
    loss = loss11.reshape(())
    return quantized, indices, loss


# speedup vs baseline: 1.2599x; 1.0196x over previous
import functools

import jax
import jax.numpy as jnp
from jax import lax
from jax.experimental import pallas as pl
from jax.experimental.pallas import tpu as pltpu
from jax.experimental.pallas import tpu_sc as plsc

_B = 8192
_C = 8192
_D = 32
_BETA = 0.25
_TB = 256
_NSTEPS = _B // _TB
_CHUNK = 2048
_RG = 64

_NC, _NS = 2, 16
_NW = _NC * _NS
_BPW = _B // _NW


def _round_bf16(v):
    u = lax.bitcast_convert_type(v, jnp.uint32)
    r = (u + jnp.uint32(0x7FFF) + ((u >> jnp.uint32(16)) & jnp.uint32(1))) \
        & jnp.uint32(0xFFFF0000)
    return lax.bitcast_convert_type(r, jnp.float32)


def _argmin_body(x_ref, codesbf_ref, xsq_ref, csq_ref, lane_ref,
                 idx_ref, loss_ref):
    i = pl.program_id(0)
    x_sq = xsq_ref[...]
    xb = x_ref[...].astype(jnp.bfloat16)
    lane = lane_ref[...]
    acc_r = acc_v = acc_i = None
    for c in range(_C // _CHUNK):
        lo = c * _CHUNK
        codes_c = codesbf_ref[lo:lo + _CHUNK, :]
        dot2 = lax.dot_general(xb, codes_c, (((1,), (1,)), ((), ())),
                               preferred_element_type=jnp.float32)
        m_parts, i_parts = [], []
        for r in range(_TB // _RG):
            rsl = slice(r * _RG, (r + 1) * _RG)
            x_sq_r = x_sq[rsl, :]
            rv = rs = None
            for s in range(_CHUNK // 128):
                sl = lo + s * 128
                d = (x_sq_r + dot2[rsl, s * 128:(s + 1) * 128]) \
                    + csq_ref[:, sl:sl + 128]
                if s == 0:
                    rv = d
                    rs = jnp.zeros_like(d)
                else:
                    lt = d < rv
                    rv = jnp.where(lt, d, rv)
                    rs = jnp.where(lt, jnp.float32(s), rs)
            rj = (rs * jnp.float32(128.0) + lane) + jnp.float32(lo)
            m_r = jnp.min(rv, axis=-1, keepdims=True)
            i_r = jnp.min(jnp.where(rv == m_r, rj, jnp.float32(_C)),
                          axis=-1, keepdims=True)
            m_parts.append(m_r)
            i_parts.append(i_r)
        m = jnp.concatenate(m_parts, axis=0)
        i_c = jnp.concatenate(i_parts, axis=0)
        if c == 0:
            acc_r, acc_v, acc_i = _round_bf16(m), m, i_c
        else:
            win = m < acc_r
            acc_r = jnp.where(win, _round_bf16(m), acc_r)
            acc_v = jnp.where(win, m, acc_v)
            acc_i = jnp.where(win, i_c, acc_i)
    idx_ref[...] = acc_i.astype(jnp.int32)
    sel = acc_v

    @pl.when(i == 0)
    def _():
        loss_ref[...] = jnp.zeros((1, 1), jnp.float32)

    loss_ref[...] += jnp.sum(sel).reshape(1, 1)

    @pl.when(i == _NSTEPS - 1)
    def _():
        loss_ref[...] = loss_ref[...] * ((1.0 + _BETA) / _B)


_argmin_kernel = pl.pallas_call(
    _argmin_body,
    grid=(_NSTEPS,),
    in_specs=[
        pl.BlockSpec((_TB, _D), lambda i: (i, 0)),
        pl.BlockSpec((_C, _D), lambda i: (0, 0)),
        pl.BlockSpec((_TB, 1), lambda i: (i, 0)),
        pl.BlockSpec((1, _C), lambda i: (0, 0)),
        pl.BlockSpec((1, 128), lambda i: (0, 0)),
    ],
    out_specs=[
        pl.BlockSpec((_TB, 1), lambda i: (i, 0)),
        pl.BlockSpec((1, 1), lambda i: (0, 0)),
    ],
    out_shape=[
        jax.ShapeDtypeStruct((_B, 1), jnp.int32),
        jax.ShapeDtypeStruct((1, 1), jnp.float32),
    ],
    compiler_params=pltpu.CompilerParams(
        dimension_semantics=("arbitrary",),
    ),
)


_DPAD = 128


@functools.cache
def _make_sc_gather():
    mesh = plsc.VectorSubcoreMesh(core_axis_name="c", subcore_axis_name="s")

    @functools.partial(
        pl.kernel,
        mesh=mesh,
        out_type=jax.ShapeDtypeStruct((_B, _D), jnp.float32),
        scratch_types=[
            pltpu.VMEM((_BPW,), jnp.int32),
            pltpu.VMEM((_BPW, _D), jnp.float32),
            pltpu.SemaphoreType.DMA,
        ],
        compiler_params=pltpu.CompilerParams(use_tc_tiling_on_sc=False),
    )
    def _sc_gather(table_hbm, idx_hbm, out_hbm, idx_v, rows_v, sem):
        wid = lax.axis_index("s") * _NC + lax.axis_index("c")
        base = wid * _BPW
        pltpu.sync_copy(idx_hbm.at[pl.ds(base, _BPW)], idx_v)
        pltpu.async_copy(table_hbm.at[idx_v], rows_v, sem).wait()
        pltpu.sync_copy(rows_v, out_hbm.at[pl.ds(base, _BPW)])

    return _sc_gather


def kernel(x, codes):
    codes2d = codes[0]
    x_sq = jnp.sum(x * x, axis=-1, keepdims=True)
    c_sq = jnp.sum(codes2d * codes2d, axis=-1)[None, :]
    codes_bf = (codes2d * jnp.float32(-2.0)).astype(jnp.bfloat16)
    lane = lax.iota(jnp.float32, 128)[None, :]
    idx2, loss11 = _argmin_kernel(x, codes_bf, x_sq, c_sq, lane)
    indices = idx2.reshape(_B)
    quantized = _make_sc_gather()(codes2d, indices)
    loss = loss11.reshape(())
    return quantized, indices, loss


# speedup vs baseline: 1.3550x; 1.0755x over previous
import functools

import jax
import jax.numpy as jnp
from jax import lax
from jax.experimental import pallas as pl
from jax.experimental.pallas import tpu as pltpu
from jax.experimental.pallas import tpu_sc as plsc

_B = 8192
_C = 8192
_D = 32
_BETA = 0.25
_TB = 512
_NSTEPS = _B // _TB
_CHUNK = 2048
_RG = 64

_NC, _NS = 2, 16
_NW = _NC * _NS
_BPW = _B // _NW


def _round_bf16(v):
    u = lax.bitcast_convert_type(v, jnp.uint32)
    r = (u + jnp.uint32(0x7FFF) + ((u >> jnp.uint32(16)) & jnp.uint32(1))) \
        & jnp.uint32(0xFFFF0000)
    return lax.bitcast_convert_type(r, jnp.float32)


def _argmin_body(x_ref, codesbf_ref, xsq_ref, csq_ref, lane_ref,
                 idx_ref, loss_ref):
    i = pl.program_id(0)
    x_sq = xsq_ref[...]
    xb = x_ref[...].astype(jnp.bfloat16)
    lane = lane_ref[...]
    acc_r = acc_v = acc_i = None
    for c in range(_C // _CHUNK):
        lo = c * _CHUNK
        codes_c = codesbf_ref[lo:lo + _CHUNK, :]
        dot2 = lax.dot_general(xb, codes_c, (((1,), (1,)), ((), ())),
                               preferred_element_type=jnp.float32)
        m_parts, i_parts = [], []
        for r in range(_TB // _RG):
            rsl = slice(r * _RG, (r + 1) * _RG)
            x_sq_r = x_sq[rsl, :]
            rv = rs = None
            for s in range(_CHUNK // 128):
                sl = lo + s * 128
                d = (x_sq_r + dot2[rsl, s * 128:(s + 1) * 128]) \
                    + csq_ref[:, sl:sl + 128]
                if s == 0:
                    rv = d
                    rs = jnp.zeros_like(d)
                else:
                    lt = d < rv
                    rv = jnp.where(lt, d, rv)
                    rs = jnp.where(lt, jnp.float32(s), rs)
            rj = (rs * jnp.float32(128.0) + lane) + jnp.float32(lo)
            m_r = jnp.min(rv, axis=-1, keepdims=True)
            i_r = jnp.min(jnp.where(rv == m_r, rj, jnp.float32(_C)),
                          axis=-1, keepdims=True)
            m_parts.append(m_r)
            i_parts.append(i_r)
        m = jnp.concatenate(m_parts, axis=0)
        i_c = jnp.concatenate(i_parts, axis=0)
        if c == 0:
            acc_r, acc_v, acc_i = _round_bf16(m), m, i_c
        else:
            win = m < acc_r
            acc_r = jnp.where(win, _round_bf16(m), acc_r)
            acc_v = jnp.where(win, m, acc_v)
            acc_i = jnp.where(win, i_c, acc_i)
    idx_ref[...] = acc_i.astype(jnp.int32)
    sel = acc_v

    @pl.when(i == 0)
    def _():
        loss_ref[...] = jnp.zeros((1, 1), jnp.float32)

    loss_ref[...] += jnp.sum(sel).reshape(1, 1)

    @pl.when(i == _NSTEPS - 1)
    def _():
        loss_ref[...] = loss_ref[...] * ((1.0 + _BETA) / _B)


_argmin_kernel = pl.pallas_call(
    _argmin_body,
    grid=(_NSTEPS,),
    in_specs=[
        pl.BlockSpec((_TB, _D), lambda i: (i, 0)),
        pl.BlockSpec((_C, _D), lambda i: (0, 0)),
        pl.BlockSpec((_TB, 1), lambda i: (i, 0)),
        pl.BlockSpec((1, _C), lambda i: (0, 0)),
        pl.BlockSpec((1, 128), lambda i: (0, 0)),
    ],
    out_specs=[
        pl.BlockSpec((_TB, 1), lambda i: (i, 0)),
        pl.BlockSpec((1, 1), lambda i: (0, 0)),
    ],
    out_shape=[
        jax.ShapeDtypeStruct((_B, 1), jnp.int32),
        jax.ShapeDtypeStruct((1, 1), jnp.float32),
    ],
    compiler_params=pltpu.CompilerParams(
        dimension_semantics=("arbitrary",),
    ),
)


_DPAD = 128


@functools.cache
def _make_sc_gather():
    mesh = plsc.VectorSubcoreMesh(core_axis_name="c", subcore_axis_name="s")

    @functools.partial(
        pl.kernel,
        mesh=mesh,
        out_type=jax.ShapeDtypeStruct((_B, _D), jnp.float32),
        scratch_types=[
            pltpu.VMEM((_BPW,), jnp.int32),
            pltpu.VMEM((_BPW, _D), jnp.float32),
            pltpu.SemaphoreType.DMA,
        ],
        compiler_params=pltpu.CompilerParams(use_tc_tiling_on_sc=False),
    )
    def _sc_gather(table_hbm, idx_hbm, out_hbm, idx_v, rows_v, sem):
        wid = lax.axis_index("s") * _NC + lax.axis_index("c")
        base = wid * _BPW
        pltpu.sync_copy(idx_hbm.at[pl.ds(base, _BPW)], idx_v)
        pltpu.async_copy(table_hbm.at[idx_v], rows_v, sem).wait()
        pltpu.sync_copy(rows_v, out_hbm.at[pl.ds(base, _BPW)])

    return _sc_gather


def kernel(x, codes):
    codes2d = codes[0]
    x_sq = jnp.sum(x * x, axis=-1, keepdims=True)
    c_sq = jnp.sum(codes2d * codes2d, axis=-1)[None, :]
    codes_bf = (codes2d * jnp.float32(-2.0)).astype(jnp.bfloat16)
    lane = lax.iota(jnp.float32, 128)[None, :]
    idx2, loss11 = _argmin_kernel(x, codes_bf, x_sq, c_sq, lane)
    indices = idx2.reshape(_B)
    quantized = _make_sc_gather()(codes2d, indices)
    loss = loss11.reshape(())
    return quantized, indices, loss


# speedup vs baseline: 1.3985x; 1.0321x over previous
import functools

import jax
import jax.numpy as jnp
from jax import lax
from jax.experimental import pallas as pl
from jax.experimental.pallas import tpu as pltpu
from jax.experimental.pallas import tpu_sc as plsc

_B = 8192
_C = 8192
_D = 32
_BETA = 0.25
_TB = 1024
_NSTEPS = _B // _TB
_CHUNK = 2048
_RG = 64

_NC, _NS = 2, 16
_NW = _NC * _NS
_BPW = _B // _NW


def _round_bf16(v):
    u = lax.bitcast_convert_type(v, jnp.uint32)
    r = (u + jnp.uint32(0x7FFF) + ((u >> jnp.uint32(16)) & jnp.uint32(1))) \
        & jnp.uint32(0xFFFF0000)
    return lax.bitcast_convert_type(r, jnp.float32)


def _argmin_body(x_ref, codesbf_ref, xsq_ref, csq_ref, lane_ref,
                 idx_ref, loss_ref):
    i = pl.program_id(0)
    x_sq = xsq_ref[...]
    xb = x_ref[...].astype(jnp.bfloat16)
    lane = lane_ref[...]
    acc_r = acc_v = acc_i = None
    for c in range(_C // _CHUNK):
        lo = c * _CHUNK
        codes_c = codesbf_ref[lo:lo + _CHUNK, :]
        dot2 = lax.dot_general(xb, codes_c, (((1,), (1,)), ((), ())),
                               preferred_element_type=jnp.float32)
        m_parts, i_parts = [], []
        for r in range(_TB // _RG):
            rsl = slice(r * _RG, (r + 1) * _RG)
            x_sq_r = x_sq[rsl, :]
            rv = rs = None
            for s in range(_CHUNK // 128):
                sl = lo + s * 128
                d = (x_sq_r + dot2[rsl, s * 128:(s + 1) * 128]) \
                    + csq_ref[:, sl:sl + 128]
                if s == 0:
                    rv = d
                    rs = jnp.zeros_like(d)
                else:
                    lt = d < rv
                    rv = jnp.where(lt, d, rv)
                    rs = jnp.where(lt, jnp.float32(s), rs)
            rj = (rs * jnp.float32(128.0) + lane) + jnp.float32(lo)
            m_r = jnp.min(rv, axis=-1, keepdims=True)
            i_r = jnp.min(jnp.where(rv == m_r, rj, jnp.float32(_C)),
                          axis=-1, keepdims=True)
            m_parts.append(m_r)
            i_parts.append(i_r)
        m = jnp.concatenate(m_parts, axis=0)
        i_c = jnp.concatenate(i_parts, axis=0)
        if c == 0:
            acc_r, acc_v, acc_i = _round_bf16(m), m, i_c
        else:
            win = m < acc_r
            acc_r = jnp.where(win, _round_bf16(m), acc_r)
            acc_v = jnp.where(win, m, acc_v)
            acc_i = jnp.where(win, i_c, acc_i)
    idx_ref[...] = acc_i.astype(jnp.int32)
    sel = acc_v

    @pl.when(i == 0)
    def _():
        loss_ref[...] = jnp.zeros((1, 1), jnp.float32)

    loss_ref[...] += jnp.sum(sel).reshape(1, 1)

    @pl.when(i == _NSTEPS - 1)
    def _():
        loss_ref[...] = loss_ref[...] * ((1.0 + _BETA) / _B)


_argmin_kernel = pl.pallas_call(
    _argmin_body,
    grid=(_NSTEPS,),
    in_specs=[
        pl.BlockSpec((_TB, _D), lambda i: (i, 0)),
        pl.BlockSpec((_C, _D), lambda i: (0, 0)),
        pl.BlockSpec((_TB, 1), lambda i: (i, 0)),
        pl.BlockSpec((1, _C), lambda i: (0, 0)),
        pl.BlockSpec((1, 128), lambda i: (0, 0)),
    ],
    out_specs=[
        pl.BlockSpec((_TB, 1), lambda i: (i, 0)),
        pl.BlockSpec((1, 1), lambda i: (0, 0)),
    ],
    out_shape=[
        jax.ShapeDtypeStruct((_B, 1), jnp.int32),
        jax.ShapeDtypeStruct((1, 1), jnp.float32),
    ],
    compiler_params=pltpu.CompilerParams(
        dimension_semantics=("arbitrary",),
    ),
)


_DPAD = 128


@functools.cache
def _make_sc_gather():
    mesh = plsc.VectorSubcoreMesh(core_axis_name="c", subcore_axis_name="s")

    @functools.partial(
        pl.kernel,
        mesh=mesh,
        out_type=jax.ShapeDtypeStruct((_B, _D), jnp.float32),
        scratch_types=[
            pltpu.VMEM((_BPW,), jnp.int32),
            pltpu.VMEM((_BPW, _D), jnp.float32),
            pltpu.SemaphoreType.DMA,
        ],
        compiler_params=pltpu.CompilerParams(use_tc_tiling_on_sc=False),
    )
    def _sc_gather(table_hbm, idx_hbm, out_hbm, idx_v, rows_v, sem):
        wid = lax.axis_index("s") * _NC + lax.axis_index("c")
        base = wid * _BPW
        pltpu.sync_copy(idx_hbm.at[pl.ds(base, _BPW)], idx_v)
        pltpu.async_copy(table_hbm.at[idx_v], rows_v, sem).wait()
        pltpu.sync_copy(rows_v, out_hbm.at[pl.ds(base, _BPW)])

    return _sc_gather


def kernel(x, codes):
    codes2d = codes[0]
    x_sq = jnp.sum(x * x, axis=-1, keepdims=True)
    c_sq = jnp.sum(codes2d * codes2d, axis=-1)[None, :]
    codes_bf = (codes2d * jnp.float32(-2.0)).astype(jnp.bfloat16)
    lane = lax.iota(jnp.float32, 128)[None, :]
    idx2, loss11 = _argmin_kernel(x, codes_bf, x_sq, c_sq, lane)
    indices = idx2.reshape(_B)
    quantized = _make_sc_gather()(codes2d, indices)
    loss = loss11.reshape(())
    return quantized, indices, loss
